# baseline (speedup 1.0000x reference)
import functools

import jax
import jax.numpy as jnp
from jax.experimental import pallas as pl
from jax.experimental.pallas import tpu as pltpu


def _fused_kernel(x_ref, wall_hbm, bias_hbm, out_ref,
                  wraw_scr, vec_scr, wall_scr, bias_scr, phi_scr, g_scr,
                  sems, *, h, w, ci):
    c = x_ref.shape[2]

    inv_ns = 1.0 / float((h // 2) * (w // 2))

    @pl.when(pl.program_id(0) == 0)
    def _load_weights():
        cw = pltpu.make_async_copy(wall_hbm, wraw_scr, sems.at[0])
        cb = pltpu.make_async_copy(bias_hbm, vec_scr, sems.at[1])
        cw.start()
        cb.start()
        cw.wait()
        cb.wait()
        vs = vec_scr[...]
        scale = (vs[3:4] * jax.lax.rsqrt(vs[6:7] + 1e-5))
        wall_scr[:c, :ci] = (wraw_scr[:ci, :].T * inv_ns).astype(jnp.bfloat16)
        wall_scr[:c, ci:2 * ci] = wraw_scr[ci:2 * ci, :].T.astype(jnp.bfloat16)
        wall_scr[:c, 2 * ci:] = wraw_scr[2 * ci:3 * ci, :].T.astype(
            jnp.bfloat16)
        wall_scr[c:, :c] = (wraw_scr[3 * ci:, :ci].T * scale).astype(
            jnp.bfloat16)
        bias_scr[0:1, :ci] = vs[0:1, :ci] * inv_ns
        bias_scr[0:1, ci:2 * ci] = vs[0:1, ci:]
        bias_scr[0:1, 2 * ci:] = vs[1:2, :ci]
        bias_scr[1:2, :c] = scale * (vs[2:3] - vs[5:6]) + vs[4:5]

    r0 = pl.ds(0, h // 2, stride=2)
    r1 = pl.ds(1, h // 2, stride=2)
    c0 = pl.ds(0, w // 2, stride=2)
    c1 = pl.ds(1, w // 2, stride=2)
    ns = (h // 2) * (w // 2)

    for img in range(x_ref.shape[0]):
        x = x_ref[img]
        xb = x.astype(jnp.bfloat16)

        tpg = (jnp.dot(xb, wall_scr[:c], preferred_element_type=jnp.float32)
               + bias_scr[0:1])
        theta_b = tpg[:, :ci].astype(jnp.bfloat16)

        phi_scr[...] = tpg[:, ci:2 * ci].reshape(h, w, ci)
        g_scr[...] = tpg[:, 2 * ci:].reshape(h, w, ci)
        pooled_phi = jnp.maximum(
            jnp.maximum(phi_scr[r0, c0], phi_scr[r0, c1]),
            jnp.maximum(phi_scr[r1, c0], phi_scr[r1, c1]))
        pooled_g = jnp.maximum(
            jnp.maximum(g_scr[r0, c0], g_scr[r0, c1]),
            jnp.maximum(g_scr[r1, c0], g_scr[r1, c1]))
        phi_t = pooled_phi.reshape(ns, ci).astype(jnp.bfloat16).T
        g = pooled_g.reshape(ns, ci).astype(jnp.bfloat16)

        f = jnp.dot(theta_b, phi_t,
                    preferred_element_type=jnp.float32)
        y = jnp.dot(f.astype(jnp.bfloat16), g,
                    preferred_element_type=jnp.float32)
        wy = (jnp.dot(y.astype(jnp.bfloat16), wall_scr[c:, :c],
                      preferred_element_type=jnp.float32)
              + bias_scr[1:2, :c])
        out_ref[img] = (wy + x).astype(out_ref.dtype)


def _nonlocal_block(x_nchw, params):
    B, C, H, W = x_nchw.shape
    N = H * W
    Ns = (H // 2) * (W // 2)
    Ci = params["w_theta"].shape[0]
    inv_ns = 1.0 / float(Ns)

    x_flat = jnp.transpose(x_nchw, (0, 2, 3, 1)).reshape(B, N, C)
    x_flat = x_flat.astype(jnp.float32)

    wraw = jnp.concatenate(
        [params["w_theta"], params["w_phi"], params["w_g"],
         jnp.pad(params["w_W"], ((0, 0), (0, C - Ci)))],
        axis=0)

    vecs = jnp.concatenate([
        params["b_theta"], params["b_phi"], params["b_g"],
        jnp.zeros((C - Ci,), jnp.float32), params["b_W"],
        params["bn_gamma"], params["bn_beta"], params["bn_mean"],
        params["bn_var"]]).reshape(7, C)

    flops = 2 * B * N * (3 * Ci * C + Ci * Ns + Ns * Ci + Ci * C)
    bytes_accessed = 2 * B * N * C * 4 + (C + Ci) * 3 * Ci * 2

    IPB = 4 if B % 4 == 0 else (2 if B % 2 == 0 else 1)

    out = pl.pallas_call(
        functools.partial(_fused_kernel, h=H, w=W, ci=Ci),
        out_shape=jax.ShapeDtypeStruct((B, N, C), jnp.float32),
        grid_spec=pltpu.PrefetchScalarGridSpec(
            num_scalar_prefetch=0,
            grid=(B // IPB,),
            in_specs=[
                pl.BlockSpec((IPB, N, C), lambda b: (b, 0, 0)),
                pl.BlockSpec(memory_space=pl.ANY),
                pl.BlockSpec(memory_space=pl.ANY),
            ],
            out_specs=pl.BlockSpec((IPB, N, C), lambda b: (b, 0, 0)),
            scratch_shapes=[
                pltpu.VMEM((3 * Ci + C, C), jnp.float32),
                pltpu.VMEM((7, C), jnp.float32),
                pltpu.VMEM((C + Ci, 3 * Ci), jnp.bfloat16),
                pltpu.VMEM((2, 3 * Ci), jnp.float32),
                pltpu.VMEM((H, W, Ci), jnp.float32),
                pltpu.VMEM((H, W, Ci), jnp.float32),
                pltpu.SemaphoreType.DMA((2,)),
            ],
        ),
        compiler_params=pltpu.CompilerParams(
            dimension_semantics=("arbitrary",),
            vmem_limit_bytes=48 * 1024 * 1024),
        cost_estimate=pl.CostEstimate(flops=flops, transcendentals=0,
                                      bytes_accessed=bytes_accessed),
    )(x_flat, wraw, vecs)

    return jnp.transpose(out.reshape(B, H, W, C), (0, 3, 1, 2))


def kernel(x, w_theta, b_theta, w_phi, b_phi, w_g, b_g, w_W, b_W,
           bn_gamma, bn_beta, bn_mean, bn_var):
    params = {
        "w_theta": w_theta, "b_theta": b_theta,
        "w_phi": w_phi, "b_phi": b_phi,
        "w_g": w_g, "b_g": b_g,
        "w_W": w_W, "b_W": b_W,
        "bn_gamma": bn_gamma, "bn_beta": bn_beta,
        "bn_mean": bn_mean, "bn_var": bn_var,
    }
    return _nonlocal_block(x, params)


# speedup vs baseline: 1.0581x; 1.0581x over previous
import functools

import jax
import jax.numpy as jnp
from jax.experimental import pallas as pl
from jax.experimental.pallas import tpu as pltpu


def _fused_kernel(x_ref, wall_hbm, bias_hbm, out_ref,
                  wraw_scr, vec_scr, wall_scr, bias_scr, phi_scr, g_scr,
                  sems, *, h, w, ci):
    c = x_ref.shape[2]

    inv_ns = 1.0 / float((h // 2) * (w // 2))

    @pl.when(pl.program_id(0) == 0)
    def _load_weights():
        cw = pltpu.make_async_copy(wall_hbm, wraw_scr, sems.at[0])
        cb = pltpu.make_async_copy(bias_hbm, vec_scr, sems.at[1])
        cw.start()
        cb.start()
        cw.wait()
        cb.wait()
        vs = vec_scr[...]
        scale = (vs[4:5] * jax.lax.rsqrt(vs[7:8] + 1e-5))
        wall_scr[:c, :ci] = (wraw_scr[:ci, :].T * inv_ns).astype(jnp.bfloat16)
        wall_scr[:c, ci:2 * ci] = wraw_scr[ci:2 * ci, :].T.astype(jnp.bfloat16)
        wall_scr[:c, 2 * ci:] = wraw_scr[2 * ci:3 * ci, :].T.astype(
            jnp.bfloat16)
        wall_scr[c:, :c] = (wraw_scr[3 * ci:, :ci].T * scale).astype(
            jnp.bfloat16)
        bias_scr[0:1, :ci] = vs[0:1, :ci] * inv_ns
        bias_scr[0:1, ci:2 * ci] = vs[1:2, :ci]
        bias_scr[0:1, 2 * ci:] = vs[2:3, :ci]
        bias_scr[1:2, :c] = scale * (vs[3:4] - vs[6:7]) + vs[5:6]

    r0 = pl.ds(0, h // 2, stride=2)
    r1 = pl.ds(1, h // 2, stride=2)
    c0 = pl.ds(0, w // 2, stride=2)
    c1 = pl.ds(1, w // 2, stride=2)
    ns = (h // 2) * (w // 2)

    for img in range(x_ref.shape[0]):
        x = x_ref[img]
        xb = x.astype(jnp.bfloat16)

        tpg = (jnp.dot(xb, wall_scr[:c], preferred_element_type=jnp.float32)
               + bias_scr[0:1])
        theta_b = tpg[:, :ci].astype(jnp.bfloat16)

        phi_scr[...] = tpg[:, ci:2 * ci].reshape(h, w, ci)
        g_scr[...] = tpg[:, 2 * ci:].reshape(h, w, ci)
        pooled_phi = jnp.maximum(
            jnp.maximum(phi_scr[r0, c0], phi_scr[r0, c1]),
            jnp.maximum(phi_scr[r1, c0], phi_scr[r1, c1]))
        pooled_g = jnp.maximum(
            jnp.maximum(g_scr[r0, c0], g_scr[r0, c1]),
            jnp.maximum(g_scr[r1, c0], g_scr[r1, c1]))
        phi_t = pooled_phi.reshape(ns, ci).astype(jnp.bfloat16).T
        g = pooled_g.reshape(ns, ci).astype(jnp.bfloat16)

        f = jnp.dot(theta_b, phi_t,
                    preferred_element_type=jnp.float32)
        y = jnp.dot(f.astype(jnp.bfloat16), g,
                    preferred_element_type=jnp.float32)
        wy = (jnp.dot(y.astype(jnp.bfloat16), wall_scr[c:, :c],
                      preferred_element_type=jnp.float32)
              + bias_scr[1:2, :c])
        out_ref[img] = (wy + x).astype(out_ref.dtype)


def _nonlocal_block(x_nchw, params):
    B, C, H, W = x_nchw.shape
    N = H * W
    Ns = (H // 2) * (W // 2)
    Ci = params["w_theta"].shape[0]
    inv_ns = 1.0 / float(Ns)

    x_flat = jnp.transpose(x_nchw, (0, 2, 3, 1)).reshape(B, N, C)
    x_flat = x_flat.astype(jnp.float32)

    wraw = jnp.concatenate(
        [params["w_theta"], params["w_phi"], params["w_g"],
         jnp.pad(params["w_W"], ((0, 0), (0, C - Ci)))],
        axis=0)

    def _row(v):
        return jnp.pad(v, (0, C - v.shape[0]))
    vecs = jnp.stack([
        _row(params["b_theta"]), _row(params["b_phi"]), _row(params["b_g"]),
        params["b_W"], params["bn_gamma"], params["bn_beta"],
        params["bn_mean"], params["bn_var"]])

    flops = 2 * B * N * (3 * Ci * C + Ci * Ns + Ns * Ci + Ci * C)
    bytes_accessed = 2 * B * N * C * 4 + (C + Ci) * 3 * Ci * 2

    IPB = 4 if B % 4 == 0 else (2 if B % 2 == 0 else 1)

    out = pl.pallas_call(
        functools.partial(_fused_kernel, h=H, w=W, ci=Ci),
        out_shape=jax.ShapeDtypeStruct((B, N, C), jnp.float32),
        grid_spec=pltpu.PrefetchScalarGridSpec(
            num_scalar_prefetch=0,
            grid=(B // IPB,),
            in_specs=[
                pl.BlockSpec((IPB, N, C), lambda b: (b, 0, 0)),
                pl.BlockSpec(memory_space=pl.ANY),
                pl.BlockSpec(memory_space=pl.ANY),
            ],
            out_specs=pl.BlockSpec((IPB, N, C), lambda b: (b, 0, 0)),
            scratch_shapes=[
                pltpu.VMEM((3 * Ci + C, C), jnp.float32),
                pltpu.VMEM((8, C), jnp.float32),
                pltpu.VMEM((C + Ci, 3 * Ci), jnp.bfloat16),
                pltpu.VMEM((2, 3 * Ci), jnp.float32),
                pltpu.VMEM((H, W, Ci), jnp.float32),
                pltpu.VMEM((H, W, Ci), jnp.float32),
                pltpu.SemaphoreType.DMA((2,)),
            ],
        ),
        compiler_params=pltpu.CompilerParams(
            dimension_semantics=("arbitrary",),
            vmem_limit_bytes=48 * 1024 * 1024),
        cost_estimate=pl.CostEstimate(flops=flops, transcendentals=0,
                                      bytes_accessed=bytes_accessed),
    )(x_flat, wraw, vecs)

    return jnp.transpose(out.reshape(B, H, W, C), (0, 3, 1, 2))


def kernel(x, w_theta, b_theta, w_phi, b_phi, w_g, b_g, w_W, b_W,
           bn_gamma, bn_beta, bn_mean, bn_var):
    params = {
        "w_theta": w_theta, "b_theta": b_theta,
        "w_phi": w_phi, "b_phi": b_phi,
        "w_g": w_g, "b_g": b_g,
        "w_W": w_W, "b_W": b_W,
        "bn_gamma": bn_gamma, "bn_beta": bn_beta,
        "bn_mean": bn_mean, "bn_var": bn_var,
    }
    return _nonlocal_block(x, params)


# speedup vs baseline: 1.0656x; 1.0071x over previous
import functools

import jax
import jax.numpy as jnp
from jax.experimental import pallas as pl
from jax.experimental.pallas import tpu as pltpu


def _fused_kernel(x_ref, wall_hbm, bias_hbm, out_ref,
                  wraw_scr, vec_scr, wall_scr, bias_scr, phi_scr, g_scr,
                  sems, *, h, w, ci):
    c = x_ref.shape[2]

    inv_ns = 1.0 / float((h // 2) * (w // 2))

    @pl.when(pl.program_id(0) == 0)
    def _load_weights():
        cw = pltpu.make_async_copy(wall_hbm, wraw_scr, sems.at[0])
        cb = pltpu.make_async_copy(bias_hbm, vec_scr, sems.at[1])
        cw.start()
        cb.start()
        cw.wait()
        cb.wait()
        vs = vec_scr[...]
        scale = (vs[4:5] * jax.lax.rsqrt(vs[7:8] + 1e-5))
        wall_scr[:c, :ci] = (wraw_scr[:ci, :].T * inv_ns).astype(jnp.bfloat16)
        wall_scr[:c, ci:2 * ci] = wraw_scr[ci:2 * ci, :].T.astype(jnp.bfloat16)
        wall_scr[:c, 2 * ci:] = wraw_scr[2 * ci:3 * ci, :].T.astype(
            jnp.bfloat16)
        wall_scr[c:, :c] = (wraw_scr[3 * ci:, :ci].T * scale).astype(
            jnp.bfloat16)
        bias_scr[0:1, :ci] = vs[0:1, :ci] * inv_ns
        bias_scr[0:1, ci:2 * ci] = vs[1:2, :ci]
        bias_scr[0:1, 2 * ci:] = vs[2:3, :ci]
        bias_scr[1:2, :c] = scale * (vs[3:4] - vs[6:7]) + vs[5:6]

    r0 = pl.ds(0, h // 2, stride=2)
    r1 = pl.ds(1, h // 2, stride=2)
    c0 = pl.ds(0, w // 2, stride=2)
    c1 = pl.ds(1, w // 2, stride=2)
    ns = (h // 2) * (w // 2)

    for img in range(x_ref.shape[0]):
        x = x_ref[img]
        xb = x.astype(jnp.bfloat16)

        tpg = (jnp.dot(xb, wall_scr[:c], preferred_element_type=jnp.float32)
               + bias_scr[0:1])
        theta_b = tpg[:, :ci].astype(jnp.bfloat16)

        phi_scr[...] = tpg[:, ci:2 * ci].reshape(h, w, ci)
        g_scr[...] = tpg[:, 2 * ci:].reshape(h, w, ci)
        pooled_phi = jnp.maximum(
            jnp.maximum(phi_scr[r0, c0], phi_scr[r0, c1]),
            jnp.maximum(phi_scr[r1, c0], phi_scr[r1, c1]))
        pooled_g = jnp.maximum(
            jnp.maximum(g_scr[r0, c0], g_scr[r0, c1]),
            jnp.maximum(g_scr[r1, c0], g_scr[r1, c1]))
        phi_t = pooled_phi.reshape(ns, ci).astype(jnp.bfloat16).T
        g = pooled_g.reshape(ns, ci).astype(jnp.bfloat16)

        f = jnp.dot(theta_b, phi_t,
                    preferred_element_type=jnp.float32)
        y = jnp.dot(f.astype(jnp.bfloat16), g,
                    preferred_element_type=jnp.float32)
        wy = (jnp.dot(y.astype(jnp.bfloat16), wall_scr[c:, :c],
                      preferred_element_type=jnp.float32)
              + bias_scr[1:2, :c])
        out_ref[img] = (wy + x).astype(out_ref.dtype)


def _nonlocal_block(x_nchw, params):
    B, C, H, W = x_nchw.shape
    N = H * W
    Ns = (H // 2) * (W // 2)
    Ci = params["w_theta"].shape[0]

    x_flat = jnp.transpose(x_nchw, (0, 2, 3, 1)).reshape(B, N, C)
    x_flat = x_flat.astype(jnp.float32)

    wraw = jnp.concatenate(
        [params["w_theta"], params["w_phi"], params["w_g"],
         jnp.pad(params["w_W"], ((0, 0), (0, C - Ci)))],
        axis=0)

    def _row(v):
        return jnp.pad(v, (0, C - v.shape[0]))
    vecs = jnp.stack([
        _row(params["b_theta"]), _row(params["b_phi"]), _row(params["b_g"]),
        params["b_W"], params["bn_gamma"], params["bn_beta"],
        params["bn_mean"], params["bn_var"]])

    flops = 2 * B * N * (3 * Ci * C + Ci * Ns + Ns * Ci + Ci * C)
    bytes_accessed = 2 * B * N * C * 4 + (C + Ci) * 3 * Ci * 2

    IPB = 4 if B % 4 == 0 else (2 if B % 2 == 0 else 1)

    out = pl.pallas_call(
        functools.partial(_fused_kernel, h=H, w=W, ci=Ci),
        out_shape=jax.ShapeDtypeStruct((B, N, C), jnp.float32),
        grid_spec=pltpu.PrefetchScalarGridSpec(
            num_scalar_prefetch=0,
            grid=(B // IPB,),
            in_specs=[
                pl.BlockSpec((IPB, N, C), lambda b: (b, 0, 0)),
                pl.BlockSpec(memory_space=pl.ANY),
                pl.BlockSpec(memory_space=pl.ANY),
            ],
            out_specs=pl.BlockSpec((IPB, N, C), lambda b: (b, 0, 0)),
            scratch_shapes=[
                pltpu.VMEM((3 * Ci + C, C), jnp.float32),
                pltpu.VMEM((8, C), jnp.float32),
                pltpu.VMEM((C + Ci, 3 * Ci), jnp.bfloat16),
                pltpu.VMEM((2, 3 * Ci), jnp.float32),
                pltpu.VMEM((H, W, Ci), jnp.float32),
                pltpu.VMEM((H, W, Ci), jnp.float32),
                pltpu.SemaphoreType.DMA((2,)),
            ],
        ),
        compiler_params=pltpu.CompilerParams(
            dimension_semantics=("arbitrary",),
            vmem_limit_bytes=48 * 1024 * 1024),
        cost_estimate=pl.CostEstimate(flops=flops, transcendentals=0,
                                      bytes_accessed=bytes_accessed),
    )(x_flat, wraw, vecs)

    return jnp.transpose(out.reshape(B, H, W, C), (0, 3, 1, 2))


def kernel(x, w_theta, b_theta, w_phi, b_phi, w_g, b_g, w_W, b_W,
           bn_gamma, bn_beta, bn_mean, bn_var):
    params = {
        "w_theta": w_theta, "b_theta": b_theta,
        "w_phi": w_phi, "b_phi": b_phi,
        "w_g": w_g, "b_g": b_g,
        "w_W": w_W, "b_W": b_W,
        "bn_gamma": bn_gamma, "bn_beta": bn_beta,
        "bn_mean": bn_mean, "bn_var": bn_var,
    }
    return _nonlocal_block(x, params)
